# Initial kernel scaffold; baseline (speedup 1.0000x reference)
import jax
import jax.numpy as jnp
from jax.experimental import pallas as pl


def kernel(x, edge_index, Wl1, bl1, Wr1, Wl2, bl2, Wr2, Wl3, bl3, Wr3):
    raise NotImplementedError("write your pallas kernel here")


# speedup vs baseline: 2.7466x; 2.7466x over previous
import functools

import jax
import jax.numpy as jnp
from jax import lax
from jax.experimental import pallas as pl
from jax.experimental.pallas import tpu as pltpu
from jax.experimental.pallas import tpu_sc as plsc

N = 10000
E = 320000
D = 128

NC = 2
NS = 16
NW = NC * NS
CHUNK = 128
CPW = 80
STG = 16
E_PAD = NW * CPW * CHUNK
N_PAD = 10112
RPT = N_PAD // NS

_mesh = plsc.VectorSubcoreMesh(core_axis_name="c", subcore_axis_name="s")


def _sc_agg_body(y_hbm, src_hbm, dst_hbm, z128, agg_out,
                 src_v, dst_v, rows_v, agg_sh):
    c = lax.axis_index("c")
    s = lax.axis_index("s")
    w = c * NS + s

    pltpu.sync_copy(z128.at[pl.ds(s * RPT, RPT)], agg_sh.at[pl.ds(s * RPT, RPT)])
    plsc.subcore_barrier()

    def stage(g, carry):
        base = w * CPW + g * STG
        pltpu.sync_copy(src_hbm.at[pl.ds(base, STG)], src_v)
        pltpu.sync_copy(dst_hbm.at[pl.ds(base, STG)], dst_v)

        def chunk(j, c2):
            pltpu.sync_copy(y_hbm.at[src_v.at[j]], rows_v)
            pltpu.sync_copy(rows_v, agg_sh.at[dst_v.at[j]], add=True)
            return c2

        lax.fori_loop(0, STG, chunk, 0)
        return carry

    lax.fori_loop(0, CPW // STG, stage, 0)
    plsc.subcore_barrier()

    pltpu.sync_copy(agg_sh.at[pl.ds(s * RPT, RPT)],
                    agg_out.at[c, pl.ds(s * RPT, RPT)])


def _sc_cnt_body(ones_hbm, dst_hbm, z128, cnt_out, dst_v, rows_v, cnt_sh):
    c = lax.axis_index("c")
    s = lax.axis_index("s")
    w = c * NS + s

    pltpu.sync_copy(z128.at[pl.ds(s * RPT, RPT)], cnt_sh.at[pl.ds(s * RPT, RPT)])
    pltpu.sync_copy(ones_hbm, rows_v)
    plsc.subcore_barrier()

    def stage(g, carry):
        base = w * CPW + g * STG
        pltpu.sync_copy(dst_hbm.at[pl.ds(base, STG)], dst_v)

        def chunk(j, c2):
            pltpu.sync_copy(rows_v, cnt_sh.at[dst_v.at[j]], add=True)
            return c2

        lax.fori_loop(0, STG, chunk, 0)
        return carry

    lax.fori_loop(0, CPW // STG, stage, 0)
    plsc.subcore_barrier()

    pltpu.sync_copy(cnt_sh.at[pl.ds(s * RPT, RPT)],
                    cnt_out.at[c, pl.ds(s * RPT, RPT)])


_sc_pass = pl.kernel(
    _sc_agg_body,
    out_type=jax.ShapeDtypeStruct((NC, N_PAD, D), jnp.float32),
    mesh=_mesh,
    scratch_types=[
        pltpu.VMEM((STG, CHUNK), jnp.int32),
        pltpu.VMEM((STG, CHUNK), jnp.int32),
        pltpu.VMEM((CHUNK, D), jnp.float32),
        pltpu.VMEM_SHARED((N_PAD, D), jnp.float32),
    ],
)

_sc_counts = pl.kernel(
    _sc_cnt_body,
    out_type=jax.ShapeDtypeStruct((NC, N_PAD, D), jnp.float32),
    mesh=_mesh,
    scratch_types=[
        pltpu.VMEM((STG, CHUNK), jnp.int32),
        pltpu.VMEM((CHUNK, D), jnp.float32),
        pltpu.VMEM_SHARED((N_PAD, D), jnp.float32),
    ],
)


R = 1000
GRID = N // R


def _lin_body(x_ref, w_ref, o_ref):
    o_ref[...] = jnp.dot(x_ref[...], w_ref[...],
                         preferred_element_type=jnp.float32)


_linear = pl.pallas_call(
    _lin_body,
    grid=(GRID,),
    in_specs=[pl.BlockSpec((R, D), lambda i: (i, 0)),
              pl.BlockSpec((D, D), lambda i: (0, 0))],
    out_specs=pl.BlockSpec((R, D), lambda i: (i, 0)),
    out_shape=jax.ShapeDtypeStruct((N, D), jnp.float32),
)


def _mean_rows(agg_ref, cnt_ref):
    inv = 1.0 / jnp.maximum(cnt_ref[0] + cnt_ref[1], 1.0)
    return (agg_ref[0] + agg_ref[1]) * inv


def _combine_body(h_ref, agg_ref, cnt_ref, wrt_ref, bl_ref, wltn_ref,
                  h_out, y_out):
    o = (_mean_rows(agg_ref, cnt_ref) + bl_ref[...]
         + jnp.dot(h_ref[...], wrt_ref[...], preferred_element_type=jnp.float32))
    hn = jnp.maximum(o, 0.0)
    h_out[...] = hn
    y_out[...] = jnp.dot(hn, wltn_ref[...], preferred_element_type=jnp.float32)


def _combine_final_body(h_ref, agg_ref, cnt_ref, wrt_ref, bl_ref, o_ref):
    o = (_mean_rows(agg_ref, cnt_ref) + bl_ref[...]
         + jnp.dot(h_ref[...], wrt_ref[...], preferred_element_type=jnp.float32))
    mx = jnp.max(o, axis=-1, keepdims=True)
    lse = jnp.log(jnp.sum(jnp.exp(o - mx), axis=-1, keepdims=True)) + mx
    o_ref[...] = o - lse


_in_specs_combine = [
    pl.BlockSpec((R, D), lambda i: (i, 0)),
    pl.BlockSpec((NC, R, D), lambda i: (0, i, 0)),
    pl.BlockSpec((NC, R, 1), lambda i: (0, i, 0)),
    pl.BlockSpec((D, D), lambda i: (0, 0)),
    pl.BlockSpec((1, D), lambda i: (0, 0)),
]

_combine = pl.pallas_call(
    _combine_body,
    grid=(GRID,),
    in_specs=_in_specs_combine + [pl.BlockSpec((D, D), lambda i: (0, 0))],
    out_specs=(pl.BlockSpec((R, D), lambda i: (i, 0)),
               pl.BlockSpec((R, D), lambda i: (i, 0))),
    out_shape=(jax.ShapeDtypeStruct((N, D), jnp.float32),
               jax.ShapeDtypeStruct((N, D), jnp.float32)),
)

_combine_final = pl.pallas_call(
    _combine_final_body,
    grid=(GRID,),
    in_specs=_in_specs_combine,
    out_specs=pl.BlockSpec((R, D), lambda i: (i, 0)),
    out_shape=jax.ShapeDtypeStruct((N, D), jnp.float32),
)


def kernel(x, edge_index, Wl1, bl1, Wr1, Wl2, bl2, Wr2, Wl3, bl3, Wr3):
    src = edge_index[0].astype(jnp.int32)
    dst = edge_index[1].astype(jnp.int32)
    npad = E_PAD - E
    srcp = jnp.concatenate([src, jnp.zeros((npad,), jnp.int32)]).reshape(-1, CHUNK)
    dstp = jnp.concatenate([dst, jnp.full((npad,), N, jnp.int32)]).reshape(-1, CHUNK)
    z128 = jnp.zeros((N_PAD, D), jnp.float32)
    ones128 = jnp.ones((CHUNK, D), jnp.float32)

    cntp = _sc_counts(ones128, dstp, z128)
    cnt = cntp[:, :N, 0:1]

    y1 = _linear(x, Wl1.T)
    agg1 = _sc_pass(y1, srcp, dstp, z128)
    h1, y2 = _combine(x, agg1, cnt, Wr1.T, bl1.reshape(1, D), Wl2.T)
    agg2 = _sc_pass(y2, srcp, dstp, z128)
    h2, y3 = _combine(h1, agg2, cnt, Wr2.T, bl2.reshape(1, D), Wl3.T)
    agg3 = _sc_pass(y3, srcp, dstp, z128)
    return _combine_final(h2, agg3, cnt, Wr3.T, bl3.reshape(1, D))


# speedup vs baseline: 3.0393x; 1.1066x over previous
import functools

import jax
import jax.numpy as jnp
from jax import lax
from jax.experimental import pallas as pl
from jax.experimental.pallas import tpu as pltpu
from jax.experimental.pallas import tpu_sc as plsc

N = 10000
E = 320000
D = 128

NC = 2
NS = 16
NW = NC * NS
CHUNK = 128
CPW = 80
STG = 16
E_PAD = NW * CPW * CHUNK
N_PAD = 10112
RPT = N_PAD // NS

_mesh = plsc.VectorSubcoreMesh(core_axis_name="c", subcore_axis_name="s")


def _sc_agg_body(y_hbm, src_hbm, dst_hbm, z128, agg_out,
                 src_v, dst_v, rows_v, gsem, agg_sh):
    c = lax.axis_index("c")
    s = lax.axis_index("s")
    w = c * NS + s

    pltpu.sync_copy(z128.at[pl.ds(s * RPT, RPT)], agg_sh.at[pl.ds(s * RPT, RPT)])
    plsc.subcore_barrier()

    def gather_start(j, b):
        pltpu.async_copy(y_hbm.at[src_v.at[j]], rows_v.at[b], gsem.at[b])

    def gather_wait(j, b):
        pltpu.make_async_copy(y_hbm.at[src_v.at[j]], rows_v.at[b],
                              gsem.at[b]).wait()

    def scatter(j, b):
        pltpu.sync_copy(rows_v.at[b], agg_sh.at[dst_v.at[j]], add=True)

    def stage(g, carry):
        base = w * CPW + g * STG
        pltpu.sync_copy(src_hbm.at[pl.ds(base, STG)], src_v)
        pltpu.sync_copy(dst_hbm.at[pl.ds(base, STG)], dst_v)

        gather_start(0, 0)

        def pair(p, c2):
            j0 = 2 * p
            gather_wait(j0, 0)
            gather_start(j0 + 1, 1)
            scatter(j0, 0)
            gather_wait(j0 + 1, 1)

            @pl.when(p < STG // 2 - 1)
            def _():
                gather_start(j0 + 2, 0)

            scatter(j0 + 1, 1)
            return c2

        lax.fori_loop(0, STG // 2, pair, 0)
        return carry

    lax.fori_loop(0, CPW // STG, stage, 0)
    plsc.subcore_barrier()

    pltpu.sync_copy(agg_sh.at[pl.ds(s * RPT, RPT)],
                    agg_out.at[c, pl.ds(s * RPT, RPT)])


def _sc_cnt_body(ones_hbm, dst_hbm, z128, cnt_out, dst_v, rows_v, cnt_sh):
    c = lax.axis_index("c")
    s = lax.axis_index("s")
    w = c * NS + s

    pltpu.sync_copy(z128.at[pl.ds(s * RPT, RPT)], cnt_sh.at[pl.ds(s * RPT, RPT)])
    pltpu.sync_copy(ones_hbm, rows_v)
    plsc.subcore_barrier()

    def stage(g, carry):
        base = w * CPW + g * STG
        pltpu.sync_copy(dst_hbm.at[pl.ds(base, STG)], dst_v)

        def chunk(j, c2):
            pltpu.sync_copy(rows_v, cnt_sh.at[dst_v.at[j]], add=True)
            return c2

        lax.fori_loop(0, STG, chunk, 0)
        return carry

    lax.fori_loop(0, CPW // STG, stage, 0)
    plsc.subcore_barrier()

    pltpu.sync_copy(cnt_sh.at[pl.ds(s * RPT, RPT)],
                    cnt_out.at[c, pl.ds(s * RPT, RPT)])


_sc_pass = pl.kernel(
    _sc_agg_body,
    out_type=jax.ShapeDtypeStruct((NC, N_PAD, D), jnp.float32),
    mesh=_mesh,
    scratch_types=[
        pltpu.VMEM((STG, CHUNK), jnp.int32),
        pltpu.VMEM((STG, CHUNK), jnp.int32),
        pltpu.VMEM((2, CHUNK, D), jnp.float32),
        pltpu.SemaphoreType.DMA((2,)),
        pltpu.VMEM_SHARED((N_PAD, D), jnp.float32),
    ],
)

_sc_counts = pl.kernel(
    _sc_cnt_body,
    out_type=jax.ShapeDtypeStruct((NC, N_PAD, D), jnp.float32),
    mesh=_mesh,
    scratch_types=[
        pltpu.VMEM((STG, CHUNK), jnp.int32),
        pltpu.VMEM((CHUNK, D), jnp.float32),
        pltpu.VMEM_SHARED((N_PAD, D), jnp.float32),
    ],
)


R = 1000
GRID = N // R


def _lin_body(x_ref, w_ref, o_ref):
    o_ref[...] = jnp.dot(x_ref[...], w_ref[...],
                         preferred_element_type=jnp.float32)


_linear = pl.pallas_call(
    _lin_body,
    grid=(GRID,),
    in_specs=[pl.BlockSpec((R, D), lambda i: (i, 0)),
              pl.BlockSpec((D, D), lambda i: (0, 0))],
    out_specs=pl.BlockSpec((R, D), lambda i: (i, 0)),
    out_shape=jax.ShapeDtypeStruct((N, D), jnp.float32),
)


def _mean_rows(agg_ref, cnt_ref):
    inv = 1.0 / jnp.maximum(cnt_ref[0] + cnt_ref[1], 1.0)
    return (agg_ref[0] + agg_ref[1]) * inv


def _combine_body(h_ref, agg_ref, cnt_ref, wrt_ref, bl_ref, wltn_ref,
                  h_out, y_out):
    o = (_mean_rows(agg_ref, cnt_ref) + bl_ref[...]
         + jnp.dot(h_ref[...], wrt_ref[...], preferred_element_type=jnp.float32))
    hn = jnp.maximum(o, 0.0)
    h_out[...] = hn
    y_out[...] = jnp.dot(hn, wltn_ref[...], preferred_element_type=jnp.float32)


def _combine_final_body(h_ref, agg_ref, cnt_ref, wrt_ref, bl_ref, o_ref):
    o = (_mean_rows(agg_ref, cnt_ref) + bl_ref[...]
         + jnp.dot(h_ref[...], wrt_ref[...], preferred_element_type=jnp.float32))
    mx = jnp.max(o, axis=-1, keepdims=True)
    lse = jnp.log(jnp.sum(jnp.exp(o - mx), axis=-1, keepdims=True)) + mx
    o_ref[...] = o - lse


_in_specs_combine = [
    pl.BlockSpec((R, D), lambda i: (i, 0)),
    pl.BlockSpec((NC, R, D), lambda i: (0, i, 0)),
    pl.BlockSpec((NC, R, 1), lambda i: (0, i, 0)),
    pl.BlockSpec((D, D), lambda i: (0, 0)),
    pl.BlockSpec((1, D), lambda i: (0, 0)),
]

_combine = pl.pallas_call(
    _combine_body,
    grid=(GRID,),
    in_specs=_in_specs_combine + [pl.BlockSpec((D, D), lambda i: (0, 0))],
    out_specs=(pl.BlockSpec((R, D), lambda i: (i, 0)),
               pl.BlockSpec((R, D), lambda i: (i, 0))),
    out_shape=(jax.ShapeDtypeStruct((N, D), jnp.float32),
               jax.ShapeDtypeStruct((N, D), jnp.float32)),
)

_combine_final = pl.pallas_call(
    _combine_final_body,
    grid=(GRID,),
    in_specs=_in_specs_combine,
    out_specs=pl.BlockSpec((R, D), lambda i: (i, 0)),
    out_shape=jax.ShapeDtypeStruct((N, D), jnp.float32),
)


def kernel(x, edge_index, Wl1, bl1, Wr1, Wl2, bl2, Wr2, Wl3, bl3, Wr3):
    src = edge_index[0].astype(jnp.int32)
    dst = edge_index[1].astype(jnp.int32)
    npad = E_PAD - E
    srcp = jnp.concatenate([src, jnp.zeros((npad,), jnp.int32)]).reshape(-1, CHUNK)
    dstp = jnp.concatenate([dst, jnp.full((npad,), N, jnp.int32)]).reshape(-1, CHUNK)
    z128 = jnp.zeros((N_PAD, D), jnp.float32)
    ones128 = jnp.ones((CHUNK, D), jnp.float32)

    cntp = _sc_counts(ones128, dstp, z128)
    cnt = cntp[:, :N, 0:1]

    y1 = _linear(x, Wl1.T)
    agg1 = _sc_pass(y1, srcp, dstp, z128)
    h1, y2 = _combine(x, agg1, cnt, Wr1.T, bl1.reshape(1, D), Wl2.T)
    agg2 = _sc_pass(y2, srcp, dstp, z128)
    h2, y3 = _combine(h1, agg2, cnt, Wr2.T, bl2.reshape(1, D), Wl3.T)
    agg3 = _sc_pass(y3, srcp, dstp, z128)
    return _combine_final(h2, agg3, cnt, Wr3.T, bl3.reshape(1, D))


# speedup vs baseline: 9.3878x; 3.0888x over previous
import functools

import jax
import jax.numpy as jnp
from jax import lax
from jax.experimental import pallas as pl
from jax.experimental.pallas import tpu as pltpu
from jax.experimental.pallas import tpu_sc as plsc

N = 10000
E = 320000
D = 128

NC = 2
NS = 16
NW = NC * NS
CHUNK = 128
CPW = 80
STG = 16
E_PAD = NW * CPW * CHUNK
N_PAD = 10112
RPT = N_PAD // NS

_mesh = plsc.VectorSubcoreMesh(core_axis_name="c", subcore_axis_name="s")


def _sc_agg_body(y_hbm, src_hbm, dst_hbm, z128, agg_out,
                 src_v, dst_v, rows_v, gsem, agg_sh):
    c = lax.axis_index("c")
    s = lax.axis_index("s")
    w = c * NS + s

    pltpu.sync_copy(z128.at[pl.ds(s * RPT, RPT)], agg_sh.at[pl.ds(s * RPT, RPT)])
    plsc.subcore_barrier()

    def gather_start(j, b):
        pltpu.async_copy(y_hbm.at[src_v.at[j]], rows_v.at[b], gsem.at[b])

    def gather_wait(j, b):
        pltpu.make_async_copy(y_hbm.at[src_v.at[j]], rows_v.at[b],
                              gsem.at[b]).wait()

    def scatter(j, b):
        pltpu.sync_copy(rows_v.at[b], agg_sh.at[dst_v.at[j]], add=True)

    def stage(g, carry):
        base = w * CPW + g * STG
        pltpu.sync_copy(src_hbm.at[pl.ds(base, STG)], src_v)
        pltpu.sync_copy(dst_hbm.at[pl.ds(base, STG)], dst_v)

        gather_start(0, 0)

        def pair(p, c2):
            j0 = 2 * p
            gather_wait(j0, 0)
            gather_start(j0 + 1, 1)
            scatter(j0, 0)
            gather_wait(j0 + 1, 1)

            @pl.when(p < STG // 2 - 1)
            def _():
                gather_start(j0 + 2, 0)

            scatter(j0 + 1, 1)
            return c2

        lax.fori_loop(0, STG // 2, pair, 0)
        return carry

    lax.fori_loop(0, CPW // STG, stage, 0)
    plsc.subcore_barrier()

    pltpu.sync_copy(agg_sh.at[pl.ds(s * RPT, RPT)],
                    agg_out.at[c, pl.ds(s * RPT, RPT)])


def _sc_cnt_body(ones_hbm, dst_hbm, z128, cnt_out, dst_v, rows_v, cnt_sh):
    c = lax.axis_index("c")
    s = lax.axis_index("s")
    w = c * NS + s

    pltpu.sync_copy(z128.at[pl.ds(s * RPT, RPT)], cnt_sh.at[pl.ds(s * RPT, RPT)])
    pltpu.sync_copy(ones_hbm, rows_v)
    plsc.subcore_barrier()

    def stage(g, carry):
        base = w * CPW + g * STG
        pltpu.sync_copy(dst_hbm.at[pl.ds(base, STG)], dst_v)

        def chunk(j, c2):
            pltpu.sync_copy(rows_v, cnt_sh.at[dst_v.at[j]], add=True)
            return c2

        lax.fori_loop(0, STG, chunk, 0)
        return carry

    lax.fori_loop(0, CPW // STG, stage, 0)
    plsc.subcore_barrier()

    pltpu.sync_copy(cnt_sh.at[pl.ds(s * RPT, RPT)],
                    cnt_out.at[c, pl.ds(s * RPT, RPT)])


_sc_pass = pl.kernel(
    _sc_agg_body,
    out_type=jax.ShapeDtypeStruct((NC, N_PAD, D), jnp.float32),
    mesh=_mesh,
    scratch_types=[
        pltpu.VMEM((STG, CHUNK), jnp.int32),
        pltpu.VMEM((STG, CHUNK), jnp.int32),
        pltpu.VMEM((2, CHUNK, D), jnp.float32),
        pltpu.SemaphoreType.DMA((2,)),
        pltpu.VMEM_SHARED((N_PAD, D), jnp.float32),
    ],
)

_sc_counts = pl.kernel(
    _sc_cnt_body,
    out_type=jax.ShapeDtypeStruct((NC, N_PAD, D), jnp.float32),
    mesh=_mesh,
    scratch_types=[
        pltpu.VMEM((STG, CHUNK), jnp.int32),
        pltpu.VMEM((CHUNK, D), jnp.float32),
        pltpu.VMEM_SHARED((N_PAD, D), jnp.float32),
    ],
)


R = 1000
GRID = N // R


def _lin_body(x_ref, w_ref, o_ref):
    o_ref[...] = jnp.dot(x_ref[...], w_ref[...],
                         preferred_element_type=jnp.float32)


_linear = pl.pallas_call(
    _lin_body,
    grid=(GRID,),
    in_specs=[pl.BlockSpec((R, D), lambda i: (i, 0)),
              pl.BlockSpec((D, D), lambda i: (0, 0))],
    out_specs=pl.BlockSpec((R, D), lambda i: (i, 0)),
    out_shape=jax.ShapeDtypeStruct((N, D), jnp.float32),
)


def _mean_rows(agg_ref, cnt_ref):
    inv = 1.0 / jnp.maximum(cnt_ref[0] + cnt_ref[1], 1.0)
    return (agg_ref[0] + agg_ref[1]) * inv


def _combine_body(h_ref, agg_ref, cnt_ref, wrt_ref, bl_ref, wltn_ref,
                  h_out, y_out):
    o = (_mean_rows(agg_ref, cnt_ref) + bl_ref[...]
         + jnp.dot(h_ref[...], wrt_ref[...], preferred_element_type=jnp.float32))
    hn = jnp.maximum(o, 0.0)
    h_out[...] = hn
    y_out[...] = jnp.dot(hn, wltn_ref[...], preferred_element_type=jnp.float32)


def _combine_final_body(h_ref, agg_ref, cnt_ref, wrt_ref, bl_ref, o_ref):
    o = (_mean_rows(agg_ref, cnt_ref) + bl_ref[...]
         + jnp.dot(h_ref[...], wrt_ref[...], preferred_element_type=jnp.float32))
    mx = jnp.max(o, axis=-1, keepdims=True)
    lse = jnp.log(jnp.sum(jnp.exp(o - mx), axis=-1, keepdims=True)) + mx
    o_ref[...] = o - lse


_in_specs_combine = [
    pl.BlockSpec((R, D), lambda i: (i, 0)),
    pl.BlockSpec((NC, R, D), lambda i: (0, i, 0)),
    pl.BlockSpec((NC, R, 1), lambda i: (0, i, 0)),
    pl.BlockSpec((D, D), lambda i: (0, 0)),
    pl.BlockSpec((1, D), lambda i: (0, 0)),
]

_combine = pl.pallas_call(
    _combine_body,
    grid=(GRID,),
    in_specs=_in_specs_combine + [pl.BlockSpec((D, D), lambda i: (0, 0))],
    out_specs=(pl.BlockSpec((R, D), lambda i: (i, 0)),
               pl.BlockSpec((R, D), lambda i: (i, 0))),
    out_shape=(jax.ShapeDtypeStruct((N, D), jnp.float32),
               jax.ShapeDtypeStruct((N, D), jnp.float32)),
)

_combine_final = pl.pallas_call(
    _combine_final_body,
    grid=(GRID,),
    in_specs=_in_specs_combine,
    out_specs=pl.BlockSpec((R, D), lambda i: (i, 0)),
    out_shape=jax.ShapeDtypeStruct((N, D), jnp.float32),
)


def kernel(x, edge_index, Wl1, bl1, Wr1, Wl2, bl2, Wr2, Wl3, bl3, Wr3):
    src = edge_index[0].astype(jnp.int32)
    dst = edge_index[1].astype(jnp.int32)
    npad = E_PAD - E
    pad_src = (jnp.arange(npad, dtype=jnp.int32) % N)
    pad_dst = N + (jnp.arange(npad, dtype=jnp.int32) % (N_PAD - N))
    srcp = jnp.concatenate([src, pad_src]).reshape(-1, CHUNK)
    dstp = jnp.concatenate([dst, pad_dst]).reshape(-1, CHUNK)
    z128 = jnp.zeros((N_PAD, D), jnp.float32)
    ones128 = jnp.ones((CHUNK, D), jnp.float32)

    cntp = _sc_counts(ones128, dstp, z128)
    cnt = cntp[:, :N, 0:1]

    y1 = _linear(x, Wl1.T)
    agg1 = _sc_pass(y1, srcp, dstp, z128)
    h1, y2 = _combine(x, agg1, cnt, Wr1.T, bl1.reshape(1, D), Wl2.T)
    agg2 = _sc_pass(y2, srcp, dstp, z128)
    h2, y3 = _combine(h1, agg2, cnt, Wr2.T, bl2.reshape(1, D), Wl3.T)
    agg3 = _sc_pass(y3, srcp, dstp, z128)
    return _combine_final(h2, agg3, cnt, Wr3.T, bl3.reshape(1, D))


# speedup vs baseline: 9.5249x; 1.0146x over previous
import functools

import jax
import jax.numpy as jnp
from jax import lax
from jax.experimental import pallas as pl
from jax.experimental.pallas import tpu as pltpu
from jax.experimental.pallas import tpu_sc as plsc

N = 10000
E = 320000
D = 128

NC = 2
NS = 16
NW = NC * NS
CHUNK = 128
CPW = 80
STG = 16
E_PAD = NW * CPW * CHUNK
N_PAD = 10112
RPT = N_PAD // NS

_mesh = plsc.VectorSubcoreMesh(core_axis_name="c", subcore_axis_name="s")


def _sc_agg_body(y_hbm, src_hbm, dst_hbm, z128, agg_out,
                 src_v, dst_v, rows_v, gsem, agg_sh):
    c = lax.axis_index("c")
    s = lax.axis_index("s")
    w = c * NS + s

    pltpu.sync_copy(z128.at[pl.ds(s * RPT, RPT)], agg_sh.at[pl.ds(s * RPT, RPT)])
    plsc.subcore_barrier()

    def gather_start(j, b):
        pltpu.async_copy(y_hbm.at[src_v.at[j]], rows_v.at[b], gsem.at[b])

    def gather_wait(j, b):
        pltpu.make_async_copy(y_hbm.at[src_v.at[j]], rows_v.at[b],
                              gsem.at[b]).wait()

    def scatter(j, b):
        pltpu.sync_copy(rows_v.at[b], agg_sh.at[dst_v.at[j]], add=True)

    def stage(g, carry):
        base = w * CPW + g * STG
        pltpu.sync_copy(src_hbm.at[pl.ds(base, STG)], src_v)
        pltpu.sync_copy(dst_hbm.at[pl.ds(base, STG)], dst_v)

        gather_start(0, 0)
        for p in range(STG // 2):
            j0 = 2 * p
            gather_wait(j0, 0)
            gather_start(j0 + 1, 1)
            scatter(j0, 0)
            gather_wait(j0 + 1, 1)
            if p < STG // 2 - 1:
                gather_start(j0 + 2, 0)
            scatter(j0 + 1, 1)
        return carry

    lax.fori_loop(0, CPW // STG, stage, 0)
    plsc.subcore_barrier()

    pltpu.sync_copy(agg_sh.at[pl.ds(s * RPT, RPT)],
                    agg_out.at[c, pl.ds(s * RPT, RPT)])


def _sc_cnt_body(ones_hbm, dst_hbm, z128, cnt_out, dst_v, rows_v, ssem, cnt_sh):
    c = lax.axis_index("c")
    s = lax.axis_index("s")
    w = c * NS + s

    pltpu.sync_copy(z128.at[pl.ds(s * RPT, RPT)], cnt_sh.at[pl.ds(s * RPT, RPT)])
    pltpu.sync_copy(dst_hbm.at[pl.ds(w * CPW, CPW)], dst_v)
    pltpu.sync_copy(ones_hbm, rows_v)
    plsc.subcore_barrier()

    def fire(j, carry):
        pltpu.async_copy(rows_v, cnt_sh.at[dst_v.at[j]], ssem, add=True)
        return carry

    lax.fori_loop(0, CPW, fire, 0)

    def drain(j, carry):
        pltpu.make_async_copy(rows_v, cnt_sh.at[dst_v.at[j]], ssem).wait()
        return carry

    lax.fori_loop(0, CPW, drain, 0)
    plsc.subcore_barrier()

    pltpu.sync_copy(cnt_sh.at[pl.ds(s * RPT, RPT)],
                    cnt_out.at[c, pl.ds(s * RPT, RPT)])


_sc_pass = pl.kernel(
    _sc_agg_body,
    out_type=jax.ShapeDtypeStruct((NC, N_PAD, D), jnp.float32),
    mesh=_mesh,
    scratch_types=[
        pltpu.VMEM((STG, CHUNK), jnp.int32),
        pltpu.VMEM((STG, CHUNK), jnp.int32),
        pltpu.VMEM((2, CHUNK, D), jnp.float32),
        pltpu.SemaphoreType.DMA((2,)),
        pltpu.VMEM_SHARED((N_PAD, D), jnp.float32),
    ],
)

_sc_counts = pl.kernel(
    _sc_cnt_body,
    out_type=jax.ShapeDtypeStruct((NC, N_PAD, D), jnp.float32),
    mesh=_mesh,
    scratch_types=[
        pltpu.VMEM((CPW, CHUNK), jnp.int32),
        pltpu.VMEM((CHUNK, D), jnp.float32),
        pltpu.SemaphoreType.DMA,
        pltpu.VMEM_SHARED((N_PAD, D), jnp.float32),
    ],
)


R = 1000
GRID = N // R


def _lin_body(x_ref, w_ref, o_ref):
    o_ref[...] = jnp.dot(x_ref[...], w_ref[...],
                         preferred_element_type=jnp.float32)


_linear = pl.pallas_call(
    _lin_body,
    grid=(GRID,),
    in_specs=[pl.BlockSpec((R, D), lambda i: (i, 0)),
              pl.BlockSpec((D, D), lambda i: (0, 0))],
    out_specs=pl.BlockSpec((R, D), lambda i: (i, 0)),
    out_shape=jax.ShapeDtypeStruct((N, D), jnp.float32),
)


def _mean_rows(agg_ref, cnt_ref):
    inv = 1.0 / jnp.maximum(cnt_ref[0] + cnt_ref[1], 1.0)
    return (agg_ref[0] + agg_ref[1]) * inv


def _combine_body(h_ref, agg_ref, cnt_ref, wrt_ref, bl_ref, wltn_ref,
                  h_out, y_out):
    o = (_mean_rows(agg_ref, cnt_ref) + bl_ref[...]
         + jnp.dot(h_ref[...], wrt_ref[...], preferred_element_type=jnp.float32))
    hn = jnp.maximum(o, 0.0)
    h_out[...] = hn
    y_out[...] = jnp.dot(hn, wltn_ref[...], preferred_element_type=jnp.float32)


def _combine_final_body(h_ref, agg_ref, cnt_ref, wrt_ref, bl_ref, o_ref):
    o = (_mean_rows(agg_ref, cnt_ref) + bl_ref[...]
         + jnp.dot(h_ref[...], wrt_ref[...], preferred_element_type=jnp.float32))
    mx = jnp.max(o, axis=-1, keepdims=True)
    lse = jnp.log(jnp.sum(jnp.exp(o - mx), axis=-1, keepdims=True)) + mx
    o_ref[...] = o - lse


_in_specs_combine = [
    pl.BlockSpec((R, D), lambda i: (i, 0)),
    pl.BlockSpec((NC, R, D), lambda i: (0, i, 0)),
    pl.BlockSpec((NC, R, 1), lambda i: (0, i, 0)),
    pl.BlockSpec((D, D), lambda i: (0, 0)),
    pl.BlockSpec((1, D), lambda i: (0, 0)),
]

_combine = pl.pallas_call(
    _combine_body,
    grid=(GRID,),
    in_specs=_in_specs_combine + [pl.BlockSpec((D, D), lambda i: (0, 0))],
    out_specs=(pl.BlockSpec((R, D), lambda i: (i, 0)),
               pl.BlockSpec((R, D), lambda i: (i, 0))),
    out_shape=(jax.ShapeDtypeStruct((N, D), jnp.float32),
               jax.ShapeDtypeStruct((N, D), jnp.float32)),
)

_combine_final = pl.pallas_call(
    _combine_final_body,
    grid=(GRID,),
    in_specs=_in_specs_combine,
    out_specs=pl.BlockSpec((R, D), lambda i: (i, 0)),
    out_shape=jax.ShapeDtypeStruct((N, D), jnp.float32),
)


def kernel(x, edge_index, Wl1, bl1, Wr1, Wl2, bl2, Wr2, Wl3, bl3, Wr3):
    src = edge_index[0].astype(jnp.int32)
    dst = edge_index[1].astype(jnp.int32)
    npad = E_PAD - E
    pad_src = (jnp.arange(npad, dtype=jnp.int32) % N)
    pad_dst = N + (jnp.arange(npad, dtype=jnp.int32) % (N_PAD - N))
    srcp = jnp.concatenate([src, pad_src]).reshape(-1, CHUNK)
    dstp = jnp.concatenate([dst, pad_dst]).reshape(-1, CHUNK)
    z128 = jnp.zeros((N_PAD, D), jnp.float32)
    ones128 = jnp.ones((CHUNK, D), jnp.float32)

    cntp = _sc_counts(ones128, dstp, z128)
    cnt = cntp[:, :N, 0:1]

    y1 = _linear(x, Wl1.T)
    agg1 = _sc_pass(y1, srcp, dstp, z128)
    h1, y2 = _combine(x, agg1, cnt, Wr1.T, bl1.reshape(1, D), Wl2.T)
    agg2 = _sc_pass(y2, srcp, dstp, z128)
    h2, y3 = _combine(h1, agg2, cnt, Wr2.T, bl2.reshape(1, D), Wl3.T)
    agg3 = _sc_pass(y3, srcp, dstp, z128)
    return _combine_final(h2, agg3, cnt, Wr3.T, bl3.reshape(1, D))


# speedup vs baseline: 9.6280x; 1.0108x over previous
import jax
import jax.numpy as jnp
from jax import lax
from jax.experimental import pallas as pl
from jax.experimental.pallas import tpu as pltpu
from jax.experimental.pallas import tpu_sc as plsc

N = 10000
E = 320000
D = 128

NC = 2
NS = 16
NW = NC * NS
CHUNK = 128
CPW = 80
STG = 16
E_PAD = NW * CPW * CHUNK
N_PAD = 10112
RPT = N_PAD // NS

_mesh = plsc.VectorSubcoreMesh(core_axis_name="c", subcore_axis_name="s")


def _zero_slice(z128, sh, s):
    pltpu.sync_copy(z128.at[pl.ds(s * RPT, RPT)], sh.at[pl.ds(s * RPT, RPT)])


def _agg_loop(y_hbm, src_hbm, dst_v, src_v, rows_v, gsem, agg_sh, w):
    def gather_start(g, j, b):
        pltpu.async_copy(y_hbm.at[src_v.at[j]], rows_v.at[b], gsem.at[b])

    def gather_wait(g, j, b):
        pltpu.make_async_copy(y_hbm.at[src_v.at[j]], rows_v.at[b],
                              gsem.at[b]).wait()

    def scatter(g, j, b):
        pltpu.sync_copy(rows_v.at[b], agg_sh.at[dst_v.at[g * STG + j]],
                        add=True)

    def stage(g, carry):
        pltpu.sync_copy(src_hbm.at[pl.ds(w * CPW + g * STG, STG)], src_v)
        gather_start(g, 0, 0)
        for p in range(STG // 2):
            j0 = 2 * p
            gather_wait(g, j0, 0)
            gather_start(g, j0 + 1, 1)
            scatter(g, j0, 0)
            gather_wait(g, j0 + 1, 1)
            if p < STG // 2 - 1:
                gather_start(g, j0 + 2, 0)
            scatter(g, j0 + 1, 1)
        return carry

    lax.fori_loop(0, CPW // STG, stage, 0)


def _spill(sh, out, c, s):
    pltpu.sync_copy(sh.at[pl.ds(s * RPT, RPT)], out.at[c, pl.ds(s * RPT, RPT)])


def _sc_agg_cnt_body(y_hbm, src_hbm, dst_hbm, z128, ones_hbm, agg_out,
                     cnt_out, src_v, dst_v, rows_v, gsem, ssem, agg_sh):
    c = lax.axis_index("c")
    s = lax.axis_index("s")
    w = c * NS + s

    _zero_slice(z128, agg_sh, s)
    pltpu.sync_copy(dst_hbm.at[pl.ds(w * CPW, CPW)], dst_v)
    pltpu.sync_copy(ones_hbm, rows_v.at[0])
    plsc.subcore_barrier()

    def fire(j, carry):
        pltpu.async_copy(rows_v.at[0], agg_sh.at[dst_v.at[j]], ssem, add=True)
        return carry

    lax.fori_loop(0, CPW, fire, 0)

    def drain(j, carry):
        pltpu.make_async_copy(rows_v.at[0], agg_sh.at[dst_v.at[j]],
                              ssem).wait()
        return carry

    lax.fori_loop(0, CPW, drain, 0)
    plsc.subcore_barrier()
    _spill(agg_sh, cnt_out, c, s)
    _zero_slice(z128, agg_sh, s)
    plsc.subcore_barrier()

    _agg_loop(y_hbm, src_hbm, dst_v, src_v, rows_v, gsem, agg_sh, w)
    plsc.subcore_barrier()
    _spill(agg_sh, agg_out, c, s)


def _sc_agg_body(y_hbm, src_hbm, dst_hbm, z128, agg_out,
                 src_v, dst_v, rows_v, gsem, agg_sh):
    c = lax.axis_index("c")
    s = lax.axis_index("s")
    w = c * NS + s

    _zero_slice(z128, agg_sh, s)
    pltpu.sync_copy(dst_hbm.at[pl.ds(w * CPW, CPW)], dst_v)
    plsc.subcore_barrier()

    _agg_loop(y_hbm, src_hbm, dst_v, src_v, rows_v, gsem, agg_sh, w)
    plsc.subcore_barrier()
    _spill(agg_sh, agg_out, c, s)


_sc_pass_counts = pl.kernel(
    _sc_agg_cnt_body,
    out_type=(jax.ShapeDtypeStruct((NC, N_PAD, D), jnp.float32),
              jax.ShapeDtypeStruct((NC, N_PAD, D), jnp.float32)),
    mesh=_mesh,
    scratch_types=[
        pltpu.VMEM((STG, CHUNK), jnp.int32),
        pltpu.VMEM((CPW, CHUNK), jnp.int32),
        pltpu.VMEM((2, CHUNK, D), jnp.float32),
        pltpu.SemaphoreType.DMA((2,)),
        pltpu.SemaphoreType.DMA,
        pltpu.VMEM_SHARED((N_PAD, D), jnp.float32),
    ],
)

_sc_pass = pl.kernel(
    _sc_agg_body,
    out_type=jax.ShapeDtypeStruct((NC, N_PAD, D), jnp.float32),
    mesh=_mesh,
    scratch_types=[
        pltpu.VMEM((STG, CHUNK), jnp.int32),
        pltpu.VMEM((CPW, CHUNK), jnp.int32),
        pltpu.VMEM((2, CHUNK, D), jnp.float32),
        pltpu.SemaphoreType.DMA((2,)),
        pltpu.VMEM_SHARED((N_PAD, D), jnp.float32),
    ],
)


R = 1000
GRID = N // R


def _lin_body(x_ref, w_ref, o_ref):
    o_ref[...] = jnp.dot(x_ref[...], w_ref[...],
                         preferred_element_type=jnp.float32)


_linear = pl.pallas_call(
    _lin_body,
    grid=(GRID,),
    in_specs=[pl.BlockSpec((R, D), lambda i: (i, 0)),
              pl.BlockSpec((D, D), lambda i: (0, 0))],
    out_specs=pl.BlockSpec((R, D), lambda i: (i, 0)),
    out_shape=jax.ShapeDtypeStruct((N, D), jnp.float32),
)


def _mean_rows(agg_ref, cnt_ref):
    cnt = cnt_ref[0, :, 0:1] + cnt_ref[1, :, 0:1]
    inv = 1.0 / jnp.maximum(cnt, 1.0)
    return (agg_ref[0] + agg_ref[1]) * inv


def _combine_body(h_ref, agg_ref, cnt_ref, wrt_ref, bl_ref, wltn_ref,
                  h_out, y_out):
    o = (_mean_rows(agg_ref, cnt_ref) + bl_ref[...]
         + jnp.dot(h_ref[...], wrt_ref[...], preferred_element_type=jnp.float32))
    hn = jnp.maximum(o, 0.0)
    h_out[...] = hn
    y_out[...] = jnp.dot(hn, wltn_ref[...], preferred_element_type=jnp.float32)


def _combine_final_body(h_ref, agg_ref, cnt_ref, wrt_ref, bl_ref, o_ref):
    o = (_mean_rows(agg_ref, cnt_ref) + bl_ref[...]
         + jnp.dot(h_ref[...], wrt_ref[...], preferred_element_type=jnp.float32))
    mx = jnp.max(o, axis=-1, keepdims=True)
    lse = jnp.log(jnp.sum(jnp.exp(o - mx), axis=-1, keepdims=True)) + mx
    o_ref[...] = o - lse


_in_specs_combine = [
    pl.BlockSpec((R, D), lambda i: (i, 0)),
    pl.BlockSpec((NC, R, D), lambda i: (0, i, 0)),
    pl.BlockSpec((NC, R, D), lambda i: (0, i, 0)),
    pl.BlockSpec((D, D), lambda i: (0, 0)),
    pl.BlockSpec((1, D), lambda i: (0, 0)),
]

_combine = pl.pallas_call(
    _combine_body,
    grid=(GRID,),
    in_specs=_in_specs_combine + [pl.BlockSpec((D, D), lambda i: (0, 0))],
    out_specs=(pl.BlockSpec((R, D), lambda i: (i, 0)),
               pl.BlockSpec((R, D), lambda i: (i, 0))),
    out_shape=(jax.ShapeDtypeStruct((N, D), jnp.float32),
               jax.ShapeDtypeStruct((N, D), jnp.float32)),
)

_combine_final = pl.pallas_call(
    _combine_final_body,
    grid=(GRID,),
    in_specs=_in_specs_combine,
    out_specs=pl.BlockSpec((R, D), lambda i: (i, 0)),
    out_shape=jax.ShapeDtypeStruct((N, D), jnp.float32),
)


def kernel(x, edge_index, Wl1, bl1, Wr1, Wl2, bl2, Wr2, Wl3, bl3, Wr3):
    src = edge_index[0].astype(jnp.int32)
    dst = edge_index[1].astype(jnp.int32)
    npad = E_PAD - E
    pad_src = (jnp.arange(npad, dtype=jnp.int32) % N)
    pad_dst = N + (jnp.arange(npad, dtype=jnp.int32) % (N_PAD - N))
    srcp = jnp.concatenate([src, pad_src]).reshape(-1, CHUNK)
    dstp = jnp.concatenate([dst, pad_dst]).reshape(-1, CHUNK)
    z128 = jnp.zeros((N_PAD, D), jnp.float32)
    ones128 = jnp.ones((CHUNK, D), jnp.float32)

    y1 = _linear(x, Wl1.T)
    agg1, cntp = _sc_pass_counts(y1, srcp, dstp, z128, ones128)
    h1, y2 = _combine(x, agg1, cntp, Wr1.T, bl1.reshape(1, D), Wl2.T)
    agg2 = _sc_pass(y2, srcp, dstp, z128)
    h2, y3 = _combine(h1, agg2, cntp, Wr2.T, bl2.reshape(1, D), Wl3.T)
    agg3 = _sc_pass(y3, srcp, dstp, z128)
    return _combine_final(h2, agg3, cntp, Wr3.T, bl3.reshape(1, D))


# speedup vs baseline: 9.8239x; 1.0203x over previous
import jax
import jax.numpy as jnp
from jax import lax
from jax.experimental import pallas as pl
from jax.experimental.pallas import tpu as pltpu
from jax.experimental.pallas import tpu_sc as plsc

N = 10000
E = 320000
D = 128

NC = 2
NS = 16
NW = NC * NS
CHUNK = 128
CPW = 80
STG = 16
E_PAD = NW * CPW * CHUNK
N_PAD = 10112
RPT = N_PAD // NS

_mesh = plsc.VectorSubcoreMesh(core_axis_name="c", subcore_axis_name="s")


def _zero_slice(z128, sh, s):
    pltpu.sync_copy(z128.at[pl.ds(s * RPT, RPT)], sh.at[pl.ds(s * RPT, RPT)])


H = CHUNK // 2


def _agg_loop(y_hbm, src_hbm, dst_v, src_v, rows_v, gsem, agg_sh, w):
    def gather_start(g, j, b):
        pltpu.async_copy(y_hbm.at[src_v.at[j, pl.ds(0, H)]],
                         rows_v.at[b, pl.ds(0, H)], gsem.at[b, 0])
        pltpu.async_copy(y_hbm.at[src_v.at[j, pl.ds(H, H)]],
                         rows_v.at[b, pl.ds(H, H)], gsem.at[b, 1])

    def gather_wait(g, j, b):
        pltpu.make_async_copy(y_hbm.at[src_v.at[j, pl.ds(0, H)]],
                              rows_v.at[b, pl.ds(0, H)], gsem.at[b, 0]).wait()
        pltpu.make_async_copy(y_hbm.at[src_v.at[j, pl.ds(H, H)]],
                              rows_v.at[b, pl.ds(H, H)], gsem.at[b, 1]).wait()

    def scatter(g, j, b):
        pltpu.sync_copy(rows_v.at[b], agg_sh.at[dst_v.at[g * STG + j]],
                        add=True)

    def stage(g, carry):
        pltpu.sync_copy(src_hbm.at[pl.ds(w * CPW + g * STG, STG)], src_v)
        gather_start(g, 0, 0)
        for p in range(STG // 2):
            j0 = 2 * p
            gather_wait(g, j0, 0)
            gather_start(g, j0 + 1, 1)
            scatter(g, j0, 0)
            gather_wait(g, j0 + 1, 1)
            if p < STG // 2 - 1:
                gather_start(g, j0 + 2, 0)
            scatter(g, j0 + 1, 1)
        return carry

    lax.fori_loop(0, CPW // STG, stage, 0)


def _spill(sh, out, c, s):
    pltpu.sync_copy(sh.at[pl.ds(s * RPT, RPT)], out.at[c, pl.ds(s * RPT, RPT)])


def _sc_agg_cnt_body(y_hbm, src_hbm, dst_hbm, z128, ones_hbm, agg_out,
                     cnt_out, src_v, dst_v, rows_v, gsem, ssem, agg_sh):
    c = lax.axis_index("c")
    s = lax.axis_index("s")
    w = c * NS + s

    _zero_slice(z128, agg_sh, s)
    pltpu.sync_copy(dst_hbm.at[pl.ds(w * CPW, CPW)], dst_v)
    pltpu.sync_copy(ones_hbm, rows_v.at[0])
    plsc.subcore_barrier()

    def fire(j, carry):
        pltpu.async_copy(rows_v.at[0], agg_sh.at[dst_v.at[j]], ssem, add=True)
        return carry

    lax.fori_loop(0, CPW, fire, 0)

    def drain(j, carry):
        pltpu.make_async_copy(rows_v.at[0], agg_sh.at[dst_v.at[j]],
                              ssem).wait()
        return carry

    lax.fori_loop(0, CPW, drain, 0)
    plsc.subcore_barrier()
    _spill(agg_sh, cnt_out, c, s)
    _zero_slice(z128, agg_sh, s)
    plsc.subcore_barrier()

    _agg_loop(y_hbm, src_hbm, dst_v, src_v, rows_v, gsem, agg_sh, w)
    plsc.subcore_barrier()
    _spill(agg_sh, agg_out, c, s)


def _sc_agg_body(y_hbm, src_hbm, dst_hbm, z128, agg_out,
                 src_v, dst_v, rows_v, gsem, agg_sh):
    c = lax.axis_index("c")
    s = lax.axis_index("s")
    w = c * NS + s

    _zero_slice(z128, agg_sh, s)
    pltpu.sync_copy(dst_hbm.at[pl.ds(w * CPW, CPW)], dst_v)
    plsc.subcore_barrier()

    _agg_loop(y_hbm, src_hbm, dst_v, src_v, rows_v, gsem, agg_sh, w)
    plsc.subcore_barrier()
    _spill(agg_sh, agg_out, c, s)


_sc_pass_counts = pl.kernel(
    _sc_agg_cnt_body,
    out_type=(jax.ShapeDtypeStruct((NC, N_PAD, D), jnp.float32),
              jax.ShapeDtypeStruct((NC, N_PAD, D), jnp.float32)),
    mesh=_mesh,
    scratch_types=[
        pltpu.VMEM((STG, CHUNK), jnp.int32),
        pltpu.VMEM((CPW, CHUNK), jnp.int32),
        pltpu.VMEM((2, CHUNK, D), jnp.float32),
        pltpu.SemaphoreType.DMA((2, 2)),
        pltpu.SemaphoreType.DMA,
        pltpu.VMEM_SHARED((N_PAD, D), jnp.float32),
    ],
)

_sc_pass = pl.kernel(
    _sc_agg_body,
    out_type=jax.ShapeDtypeStruct((NC, N_PAD, D), jnp.float32),
    mesh=_mesh,
    scratch_types=[
        pltpu.VMEM((STG, CHUNK), jnp.int32),
        pltpu.VMEM((CPW, CHUNK), jnp.int32),
        pltpu.VMEM((2, CHUNK, D), jnp.float32),
        pltpu.SemaphoreType.DMA((2, 2)),
        pltpu.VMEM_SHARED((N_PAD, D), jnp.float32),
    ],
)


R = 1000
GRID = N // R


def _lin_body(x_ref, w_ref, o_ref):
    o_ref[...] = jnp.dot(x_ref[...], w_ref[...],
                         preferred_element_type=jnp.float32)


_linear = pl.pallas_call(
    _lin_body,
    grid=(GRID,),
    in_specs=[pl.BlockSpec((R, D), lambda i: (i, 0)),
              pl.BlockSpec((D, D), lambda i: (0, 0))],
    out_specs=pl.BlockSpec((R, D), lambda i: (i, 0)),
    out_shape=jax.ShapeDtypeStruct((N, D), jnp.float32),
)


def _mean_rows(agg_ref, cnt_ref):
    cnt = cnt_ref[0, :, 0:1] + cnt_ref[1, :, 0:1]
    inv = 1.0 / jnp.maximum(cnt, 1.0)
    return (agg_ref[0] + agg_ref[1]) * inv


def _combine_body(h_ref, agg_ref, cnt_ref, wrt_ref, bl_ref, wltn_ref,
                  h_out, y_out):
    o = (_mean_rows(agg_ref, cnt_ref) + bl_ref[...]
         + jnp.dot(h_ref[...], wrt_ref[...], preferred_element_type=jnp.float32))
    hn = jnp.maximum(o, 0.0)
    h_out[...] = hn
    y_out[...] = jnp.dot(hn, wltn_ref[...], preferred_element_type=jnp.float32)


def _combine_final_body(h_ref, agg_ref, cnt_ref, wrt_ref, bl_ref, o_ref):
    o = (_mean_rows(agg_ref, cnt_ref) + bl_ref[...]
         + jnp.dot(h_ref[...], wrt_ref[...], preferred_element_type=jnp.float32))
    mx = jnp.max(o, axis=-1, keepdims=True)
    lse = jnp.log(jnp.sum(jnp.exp(o - mx), axis=-1, keepdims=True)) + mx
    o_ref[...] = o - lse


_in_specs_combine = [
    pl.BlockSpec((R, D), lambda i: (i, 0)),
    pl.BlockSpec((NC, R, D), lambda i: (0, i, 0)),
    pl.BlockSpec((NC, R, D), lambda i: (0, i, 0)),
    pl.BlockSpec((D, D), lambda i: (0, 0)),
    pl.BlockSpec((1, D), lambda i: (0, 0)),
]

_combine = pl.pallas_call(
    _combine_body,
    grid=(GRID,),
    in_specs=_in_specs_combine + [pl.BlockSpec((D, D), lambda i: (0, 0))],
    out_specs=(pl.BlockSpec((R, D), lambda i: (i, 0)),
               pl.BlockSpec((R, D), lambda i: (i, 0))),
    out_shape=(jax.ShapeDtypeStruct((N, D), jnp.float32),
               jax.ShapeDtypeStruct((N, D), jnp.float32)),
)

_combine_final = pl.pallas_call(
    _combine_final_body,
    grid=(GRID,),
    in_specs=_in_specs_combine,
    out_specs=pl.BlockSpec((R, D), lambda i: (i, 0)),
    out_shape=jax.ShapeDtypeStruct((N, D), jnp.float32),
)


def kernel(x, edge_index, Wl1, bl1, Wr1, Wl2, bl2, Wr2, Wl3, bl3, Wr3):
    src = edge_index[0].astype(jnp.int32)
    dst = edge_index[1].astype(jnp.int32)
    npad = E_PAD - E
    pad_src = (jnp.arange(npad, dtype=jnp.int32) % N)
    pad_dst = N + (jnp.arange(npad, dtype=jnp.int32) % (N_PAD - N))
    srcp = jnp.concatenate([src, pad_src]).reshape(-1, CHUNK)
    dstp = jnp.concatenate([dst, pad_dst]).reshape(-1, CHUNK)
    z128 = jnp.zeros((N_PAD, D), jnp.float32)
    ones128 = jnp.ones((CHUNK, D), jnp.float32)

    y1 = _linear(x, Wl1.T)
    agg1, cntp = _sc_pass_counts(y1, srcp, dstp, z128, ones128)
    h1, y2 = _combine(x, agg1, cntp, Wr1.T, bl1.reshape(1, D), Wl2.T)
    agg2 = _sc_pass(y2, srcp, dstp, z128)
    h2, y3 = _combine(h1, agg2, cntp, Wr2.T, bl2.reshape(1, D), Wl3.T)
    agg3 = _sc_pass(y3, srcp, dstp, z128)
    return _combine_final(h2, agg3, cntp, Wr3.T, bl3.reshape(1, D))


# speedup vs baseline: 9.9632x; 1.0142x over previous
import jax
import jax.numpy as jnp
from jax import lax
from jax.experimental import pallas as pl
from jax.experimental.pallas import tpu as pltpu
from jax.experimental.pallas import tpu_sc as plsc

N = 10000
E = 320000
D = 128

NC = 2
NS = 16
NW = NC * NS
CHUNK = 128
CPW = 80
STG = 16
E_PAD = NW * CPW * CHUNK
N_PAD = 10112
RPT = N_PAD // NS

_mesh = plsc.VectorSubcoreMesh(core_axis_name="c", subcore_axis_name="s")


def _fill_zero_buf(zbuf):
    zv = jnp.zeros((16,), jnp.float32)

    def row(i, carry):
        for k in range(D // 16):
            zbuf[i, pl.ds(16 * k, 16)] = zv
        return carry

    lax.fori_loop(0, CHUNK, row, 0)


def _zero_slice(zbuf, sh, s):
    base = s * RPT
    for off, n in ((0, 128), (128, 128), (256, 128), (384, 128), (512, 120)):
        pltpu.sync_copy(zbuf.at[pl.ds(0, n)], sh.at[pl.ds(base + off, n)])


H = CHUNK // 2


def _agg_loop(y_hbm, src_hbm, dst_v, src_v, rows_v, gsem, agg_sh, w):
    def gather_start(g, j, b):
        pltpu.async_copy(y_hbm.at[src_v.at[j, pl.ds(0, H)]],
                         rows_v.at[b, pl.ds(0, H)], gsem.at[b, 0])
        pltpu.async_copy(y_hbm.at[src_v.at[j, pl.ds(H, H)]],
                         rows_v.at[b, pl.ds(H, H)], gsem.at[b, 1])

    def gather_wait(g, j, b):
        pltpu.make_async_copy(y_hbm.at[src_v.at[j, pl.ds(0, H)]],
                              rows_v.at[b, pl.ds(0, H)], gsem.at[b, 0]).wait()
        pltpu.make_async_copy(y_hbm.at[src_v.at[j, pl.ds(H, H)]],
                              rows_v.at[b, pl.ds(H, H)], gsem.at[b, 1]).wait()

    def scatter(g, j, b):
        pltpu.sync_copy(rows_v.at[b], agg_sh.at[dst_v.at[g * STG + j]],
                        add=True)

    def stage(g, carry):
        pltpu.sync_copy(src_hbm.at[pl.ds(w * CPW + g * STG, STG)], src_v)
        gather_start(g, 0, 0)
        for p in range(STG // 2):
            j0 = 2 * p
            gather_wait(g, j0, 0)
            gather_start(g, j0 + 1, 1)
            scatter(g, j0, 0)
            gather_wait(g, j0 + 1, 1)
            if p < STG // 2 - 1:
                gather_start(g, j0 + 2, 0)
            scatter(g, j0 + 1, 1)
        return carry

    lax.fori_loop(0, CPW // STG, stage, 0)


def _spill(sh, out, c, s):
    pltpu.sync_copy(sh.at[pl.ds(s * RPT, RPT)], out.at[c, pl.ds(s * RPT, RPT)])


def _sc_agg_cnt_body(y_hbm, src_hbm, dst_hbm, ones_hbm, agg_out,
                     cnt_out, src_v, dst_v, rows_v, gsem, ssem, agg_sh):
    c = lax.axis_index("c")
    s = lax.axis_index("s")
    w = c * NS + s

    _fill_zero_buf(rows_v.at[1])
    _zero_slice(rows_v.at[1], agg_sh, s)
    pltpu.sync_copy(dst_hbm.at[pl.ds(w * CPW, CPW)], dst_v)
    pltpu.sync_copy(ones_hbm, rows_v.at[0])
    plsc.subcore_barrier()

    def fire(j, carry):
        pltpu.async_copy(rows_v.at[0], agg_sh.at[dst_v.at[j]], ssem, add=True)
        return carry

    lax.fori_loop(0, CPW, fire, 0)

    def drain(j, carry):
        pltpu.make_async_copy(rows_v.at[0], agg_sh.at[dst_v.at[j]],
                              ssem).wait()
        return carry

    lax.fori_loop(0, CPW, drain, 0)
    plsc.subcore_barrier()
    _spill(agg_sh, cnt_out, c, s)
    _zero_slice(rows_v.at[1], agg_sh, s)
    plsc.subcore_barrier()

    _agg_loop(y_hbm, src_hbm, dst_v, src_v, rows_v, gsem, agg_sh, w)
    plsc.subcore_barrier()
    _spill(agg_sh, agg_out, c, s)


def _sc_agg_body(y_hbm, src_hbm, dst_hbm, agg_out,
                 src_v, dst_v, rows_v, gsem, agg_sh):
    c = lax.axis_index("c")
    s = lax.axis_index("s")
    w = c * NS + s

    _fill_zero_buf(rows_v.at[1])
    _zero_slice(rows_v.at[1], agg_sh, s)
    pltpu.sync_copy(dst_hbm.at[pl.ds(w * CPW, CPW)], dst_v)
    plsc.subcore_barrier()

    _agg_loop(y_hbm, src_hbm, dst_v, src_v, rows_v, gsem, agg_sh, w)
    plsc.subcore_barrier()
    _spill(agg_sh, agg_out, c, s)


_sc_pass_counts = pl.kernel(
    _sc_agg_cnt_body,
    out_type=(jax.ShapeDtypeStruct((NC, N_PAD, D), jnp.float32),
              jax.ShapeDtypeStruct((NC, N_PAD, D), jnp.float32)),
    mesh=_mesh,
    scratch_types=[
        pltpu.VMEM((STG, CHUNK), jnp.int32),
        pltpu.VMEM((CPW, CHUNK), jnp.int32),
        pltpu.VMEM((2, CHUNK, D), jnp.float32),
        pltpu.SemaphoreType.DMA((2, 2)),
        pltpu.SemaphoreType.DMA,
        pltpu.VMEM_SHARED((N_PAD, D), jnp.float32),
    ],
)

_sc_pass = pl.kernel(
    _sc_agg_body,
    out_type=jax.ShapeDtypeStruct((NC, N_PAD, D), jnp.float32),
    mesh=_mesh,
    scratch_types=[
        pltpu.VMEM((STG, CHUNK), jnp.int32),
        pltpu.VMEM((CPW, CHUNK), jnp.int32),
        pltpu.VMEM((2, CHUNK, D), jnp.float32),
        pltpu.SemaphoreType.DMA((2, 2)),
        pltpu.VMEM_SHARED((N_PAD, D), jnp.float32),
    ],
)


R = 1000
GRID = N // R


def _lin_body(x_ref, w_ref, o_ref):
    o_ref[...] = jnp.dot(x_ref[...], w_ref[...],
                         preferred_element_type=jnp.float32)


_linear = pl.pallas_call(
    _lin_body,
    grid=(GRID,),
    in_specs=[pl.BlockSpec((R, D), lambda i: (i, 0)),
              pl.BlockSpec((D, D), lambda i: (0, 0))],
    out_specs=pl.BlockSpec((R, D), lambda i: (i, 0)),
    out_shape=jax.ShapeDtypeStruct((N, D), jnp.float32),
)


def _mean_rows(agg_ref, cnt_ref):
    cnt = cnt_ref[0, :, 0:1] + cnt_ref[1, :, 0:1]
    inv = 1.0 / jnp.maximum(cnt, 1.0)
    return (agg_ref[0] + agg_ref[1]) * inv


def _combine_body(h_ref, agg_ref, cnt_ref, wrt_ref, bl_ref, wltn_ref,
                  h_out, y_out):
    o = (_mean_rows(agg_ref, cnt_ref) + bl_ref[...]
         + jnp.dot(h_ref[...], wrt_ref[...], preferred_element_type=jnp.float32))
    hn = jnp.maximum(o, 0.0)
    h_out[...] = hn
    y_out[...] = jnp.dot(hn, wltn_ref[...], preferred_element_type=jnp.float32)


def _combine_final_body(h_ref, agg_ref, cnt_ref, wrt_ref, bl_ref, o_ref):
    o = (_mean_rows(agg_ref, cnt_ref) + bl_ref[...]
         + jnp.dot(h_ref[...], wrt_ref[...], preferred_element_type=jnp.float32))
    mx = jnp.max(o, axis=-1, keepdims=True)
    lse = jnp.log(jnp.sum(jnp.exp(o - mx), axis=-1, keepdims=True)) + mx
    o_ref[...] = o - lse


_in_specs_combine = [
    pl.BlockSpec((R, D), lambda i: (i, 0)),
    pl.BlockSpec((NC, R, D), lambda i: (0, i, 0)),
    pl.BlockSpec((NC, R, D), lambda i: (0, i, 0)),
    pl.BlockSpec((D, D), lambda i: (0, 0)),
    pl.BlockSpec((1, D), lambda i: (0, 0)),
]

_combine = pl.pallas_call(
    _combine_body,
    grid=(GRID,),
    in_specs=_in_specs_combine + [pl.BlockSpec((D, D), lambda i: (0, 0))],
    out_specs=(pl.BlockSpec((R, D), lambda i: (i, 0)),
               pl.BlockSpec((R, D), lambda i: (i, 0))),
    out_shape=(jax.ShapeDtypeStruct((N, D), jnp.float32),
               jax.ShapeDtypeStruct((N, D), jnp.float32)),
)

_combine_final = pl.pallas_call(
    _combine_final_body,
    grid=(GRID,),
    in_specs=_in_specs_combine,
    out_specs=pl.BlockSpec((R, D), lambda i: (i, 0)),
    out_shape=jax.ShapeDtypeStruct((N, D), jnp.float32),
)


def kernel(x, edge_index, Wl1, bl1, Wr1, Wl2, bl2, Wr2, Wl3, bl3, Wr3):
    src = edge_index[0].astype(jnp.int32)
    dst = edge_index[1].astype(jnp.int32)
    npad = E_PAD - E
    pad_src = (jnp.arange(npad, dtype=jnp.int32) % N)
    pad_dst = N + (jnp.arange(npad, dtype=jnp.int32) % (N_PAD - N))
    srcp = jnp.concatenate([src, pad_src]).reshape(-1, CHUNK)
    dstp = jnp.concatenate([dst, pad_dst]).reshape(-1, CHUNK)
    ones128 = jnp.ones((CHUNK, D), jnp.float32)

    y1 = _linear(x, Wl1.T)
    agg1, cntp = _sc_pass_counts(y1, srcp, dstp, ones128)
    h1, y2 = _combine(x, agg1, cntp, Wr1.T, bl1.reshape(1, D), Wl2.T)
    agg2 = _sc_pass(y2, srcp, dstp)
    h2, y3 = _combine(h1, agg2, cntp, Wr2.T, bl2.reshape(1, D), Wl3.T)
    agg3 = _sc_pass(y3, srcp, dstp)
    return _combine_final(h2, agg3, cntp, Wr3.T, bl3.reshape(1, D))


# speedup vs baseline: 10.2367x; 1.0275x over previous
import jax
import jax.numpy as jnp
from jax import lax
from jax.experimental import pallas as pl
from jax.experimental.pallas import tpu as pltpu
from jax.experimental.pallas import tpu_sc as plsc

N = 10000
E = 320000
D = 128

NC = 2
NS = 16
NW = NC * NS
CHUNK = 128
CPW = 80
STG = 8
E_PAD = NW * CPW * CHUNK
N_PAD = 10112
RPT = N_PAD // NS

_mesh = plsc.VectorSubcoreMesh(core_axis_name="c", subcore_axis_name="s")


def _fill_zero_buf(zbuf):
    zv = jnp.zeros((16,), jnp.float32)

    def row(i, carry):
        for k in range(D // 16):
            zbuf[i, pl.ds(16 * k, 16)] = zv
        return carry

    lax.fori_loop(0, CHUNK, row, 0)


def _zero_slice(zbuf, sh, s):
    base = s * RPT
    for off, n in ((0, 128), (128, 128), (256, 128), (384, 128), (512, 120)):
        pltpu.sync_copy(zbuf.at[pl.ds(0, n)], sh.at[pl.ds(base + off, n)])


H = CHUNK // 2


BPB = 2 * STG
NBODY = CPW // BPB


def _agg_loop(y_hbm, src_hbm, dst_v, src_v, rows_v, gsem, stg_sem, agg_sh, w):
    def gather_start(h, r, b):
        pltpu.async_copy(y_hbm.at[src_v.at[h, r, pl.ds(0, H)]],
                         rows_v.at[b, pl.ds(0, H)], gsem.at[b, 0])
        pltpu.async_copy(y_hbm.at[src_v.at[h, r, pl.ds(H, H)]],
                         rows_v.at[b, pl.ds(H, H)], gsem.at[b, 1])

    def gather_wait(h, r, b):
        pltpu.make_async_copy(y_hbm.at[src_v.at[h, r, pl.ds(0, H)]],
                              rows_v.at[b, pl.ds(0, H)], gsem.at[b, 0]).wait()
        pltpu.make_async_copy(y_hbm.at[src_v.at[h, r, pl.ds(H, H)]],
                              rows_v.at[b, pl.ds(H, H)], gsem.at[b, 1]).wait()

    def scatter(gg, j, b):
        pltpu.sync_copy(rows_v.at[b], agg_sh.at[dst_v.at[gg * BPB + j]],
                        add=True)

    def stage_issue(h, row0):
        pltpu.async_copy(src_hbm.at[pl.ds(row0, STG)], src_v.at[h],
                         stg_sem.at[h])

    def stage_wait(h):
        pltpu.make_async_copy(src_hbm.at[pl.ds(0, STG)], src_v.at[h],
                              stg_sem.at[h]).wait()

    pltpu.sync_copy(src_hbm.at[pl.ds(w * CPW, STG)], src_v.at[0])
    stage_issue(1, w * CPW + STG)
    gather_start(0, 0, 0)

    def body(gg, carry):
        for p in range(BPB // 2):
            j0, j1, j2 = 2 * p, 2 * p + 1, 2 * p + 2
            h0, r0 = divmod(j0, STG)
            h1, r1 = divmod(j1, STG)
            gather_wait(h0, r0, 0)
            gather_start(h1, r1, 1)
            scatter(gg, j0, 0)
            gather_wait(h1, r1, 1)
            if p == STG // 2 - 1:
                @pl.when(gg < NBODY - 1)
                def _():
                    stage_issue(0, w * CPW + (gg + 1) * BPB)

                stage_wait(1)
                gather_start(1, 0, 0)
            elif p == BPB // 2 - 1:
                @pl.when(gg < NBODY - 1)
                def _():
                    stage_issue(1, w * CPW + (gg + 1) * BPB + STG)
                    stage_wait(0)
                    gather_start(0, 0, 0)
            else:
                h2, r2 = divmod(j2, STG)
                gather_start(h2, r2, 0)
            scatter(gg, j1, 1)
        return carry

    lax.fori_loop(0, NBODY, body, 0)


def _spill(sh, out, c, s):
    pltpu.sync_copy(sh.at[pl.ds(s * RPT, RPT)], out.at[c, pl.ds(s * RPT, RPT)])


def _sc_agg_cnt_body(y_hbm, src_hbm, dst_hbm, ones_hbm, agg_out,
                     cnt_out, src_v, dst_v, rows_v, gsem, stg_sem, ssem,
                     agg_sh):
    c = lax.axis_index("c")
    s = lax.axis_index("s")
    w = c * NS + s

    _fill_zero_buf(rows_v.at[1])
    _zero_slice(rows_v.at[1], agg_sh, s)
    pltpu.sync_copy(dst_hbm.at[pl.ds(w * CPW, CPW)], dst_v)
    pltpu.sync_copy(ones_hbm, rows_v.at[0])
    plsc.subcore_barrier()

    def fire(j, carry):
        pltpu.async_copy(rows_v.at[0], agg_sh.at[dst_v.at[j]], ssem, add=True)
        return carry

    lax.fori_loop(0, CPW, fire, 0)

    def drain(j, carry):
        pltpu.make_async_copy(rows_v.at[0], agg_sh.at[dst_v.at[j]],
                              ssem).wait()
        return carry

    lax.fori_loop(0, CPW, drain, 0)
    plsc.subcore_barrier()
    _spill(agg_sh, cnt_out, c, s)
    _zero_slice(rows_v.at[1], agg_sh, s)
    plsc.subcore_barrier()

    _agg_loop(y_hbm, src_hbm, dst_v, src_v, rows_v, gsem, stg_sem, agg_sh, w)
    plsc.subcore_barrier()
    _spill(agg_sh, agg_out, c, s)


def _sc_agg_body(y_hbm, src_hbm, dst_hbm, agg_out,
                 src_v, dst_v, rows_v, gsem, stg_sem, agg_sh):
    c = lax.axis_index("c")
    s = lax.axis_index("s")
    w = c * NS + s

    _fill_zero_buf(rows_v.at[1])
    _zero_slice(rows_v.at[1], agg_sh, s)
    pltpu.sync_copy(dst_hbm.at[pl.ds(w * CPW, CPW)], dst_v)
    plsc.subcore_barrier()

    _agg_loop(y_hbm, src_hbm, dst_v, src_v, rows_v, gsem, stg_sem, agg_sh, w)
    plsc.subcore_barrier()
    _spill(agg_sh, agg_out, c, s)


_sc_pass_counts = pl.kernel(
    _sc_agg_cnt_body,
    out_type=(jax.ShapeDtypeStruct((NC, N_PAD, D), jnp.float32),
              jax.ShapeDtypeStruct((NC, N_PAD, D), jnp.float32)),
    mesh=_mesh,
    scratch_types=[
        pltpu.VMEM((2, STG, CHUNK), jnp.int32),
        pltpu.VMEM((CPW, CHUNK), jnp.int32),
        pltpu.VMEM((2, CHUNK, D), jnp.float32),
        pltpu.SemaphoreType.DMA((2, 2)),
        pltpu.SemaphoreType.DMA((2,)),
        pltpu.SemaphoreType.DMA,
        pltpu.VMEM_SHARED((N_PAD, D), jnp.float32),
    ],
)

_sc_pass = pl.kernel(
    _sc_agg_body,
    out_type=jax.ShapeDtypeStruct((NC, N_PAD, D), jnp.float32),
    mesh=_mesh,
    scratch_types=[
        pltpu.VMEM((2, STG, CHUNK), jnp.int32),
        pltpu.VMEM((CPW, CHUNK), jnp.int32),
        pltpu.VMEM((2, CHUNK, D), jnp.float32),
        pltpu.SemaphoreType.DMA((2, 2)),
        pltpu.SemaphoreType.DMA((2,)),
        pltpu.VMEM_SHARED((N_PAD, D), jnp.float32),
    ],
)


R = 1000
GRID = N // R


def _lin_body(x_ref, w_ref, o_ref):
    o_ref[...] = jnp.dot(x_ref[...], w_ref[...],
                         preferred_element_type=jnp.float32)


_linear = pl.pallas_call(
    _lin_body,
    grid=(GRID,),
    in_specs=[pl.BlockSpec((R, D), lambda i: (i, 0)),
              pl.BlockSpec((D, D), lambda i: (0, 0))],
    out_specs=pl.BlockSpec((R, D), lambda i: (i, 0)),
    out_shape=jax.ShapeDtypeStruct((N, D), jnp.float32),
)


def _mean_rows(agg_ref, cnt_ref):
    cnt = cnt_ref[0, :, 0:1] + cnt_ref[1, :, 0:1]
    inv = 1.0 / jnp.maximum(cnt, 1.0)
    return (agg_ref[0] + agg_ref[1]) * inv


def _combine_body(h_ref, agg_ref, cnt_ref, wrt_ref, bl_ref, wltn_ref,
                  h_out, y_out):
    o = (_mean_rows(agg_ref, cnt_ref) + bl_ref[...]
         + jnp.dot(h_ref[...], wrt_ref[...], preferred_element_type=jnp.float32))
    hn = jnp.maximum(o, 0.0)
    h_out[...] = hn
    y_out[...] = jnp.dot(hn, wltn_ref[...], preferred_element_type=jnp.float32)


def _combine_final_body(h_ref, agg_ref, cnt_ref, wrt_ref, bl_ref, o_ref):
    o = (_mean_rows(agg_ref, cnt_ref) + bl_ref[...]
         + jnp.dot(h_ref[...], wrt_ref[...], preferred_element_type=jnp.float32))
    mx = jnp.max(o, axis=-1, keepdims=True)
    lse = jnp.log(jnp.sum(jnp.exp(o - mx), axis=-1, keepdims=True)) + mx
    o_ref[...] = o - lse


_in_specs_combine = [
    pl.BlockSpec((R, D), lambda i: (i, 0)),
    pl.BlockSpec((NC, R, D), lambda i: (0, i, 0)),
    pl.BlockSpec((NC, R, D), lambda i: (0, i, 0)),
    pl.BlockSpec((D, D), lambda i: (0, 0)),
    pl.BlockSpec((1, D), lambda i: (0, 0)),
]

_combine = pl.pallas_call(
    _combine_body,
    grid=(GRID,),
    in_specs=_in_specs_combine + [pl.BlockSpec((D, D), lambda i: (0, 0))],
    out_specs=(pl.BlockSpec((R, D), lambda i: (i, 0)),
               pl.BlockSpec((R, D), lambda i: (i, 0))),
    out_shape=(jax.ShapeDtypeStruct((N, D), jnp.float32),
               jax.ShapeDtypeStruct((N, D), jnp.float32)),
)

_combine_final = pl.pallas_call(
    _combine_final_body,
    grid=(GRID,),
    in_specs=_in_specs_combine,
    out_specs=pl.BlockSpec((R, D), lambda i: (i, 0)),
    out_shape=jax.ShapeDtypeStruct((N, D), jnp.float32),
)


def kernel(x, edge_index, Wl1, bl1, Wr1, Wl2, bl2, Wr2, Wl3, bl3, Wr3):
    src = edge_index[0].astype(jnp.int32)
    dst = edge_index[1].astype(jnp.int32)
    npad = E_PAD - E
    pad_src = (jnp.arange(npad, dtype=jnp.int32) % N)
    pad_dst = N + (jnp.arange(npad, dtype=jnp.int32) % (N_PAD - N))
    srcp = jnp.concatenate([src, pad_src]).reshape(-1, CHUNK)
    dstp = jnp.concatenate([dst, pad_dst]).reshape(-1, CHUNK)
    ones128 = jnp.ones((CHUNK, D), jnp.float32)

    y1 = _linear(x, Wl1.T)
    agg1, cntp = _sc_pass_counts(y1, srcp, dstp, ones128)
    h1, y2 = _combine(x, agg1, cntp, Wr1.T, bl1.reshape(1, D), Wl2.T)
    agg2 = _sc_pass(y2, srcp, dstp)
    h2, y3 = _combine(h1, agg2, cntp, Wr2.T, bl2.reshape(1, D), Wl3.T)
    agg3 = _sc_pass(y3, srcp, dstp)
    return _combine_final(h2, agg3, cntp, Wr3.T, bl3.reshape(1, D))


# speedup vs baseline: 10.2485x; 1.0011x over previous
import jax
import jax.numpy as jnp
from jax import lax
from jax.experimental import pallas as pl
from jax.experimental.pallas import tpu as pltpu
from jax.experimental.pallas import tpu_sc as plsc

N = 10000
E = 320000
D = 128

NC = 2
NS = 16
NW = NC * NS
CHUNK = 128
CPW = 80
STG = 8
E_PAD = NW * CPW * CHUNK
N_PAD = 10112
RPT = N_PAD // NS

_mesh = plsc.VectorSubcoreMesh(core_axis_name="c", subcore_axis_name="s")


def _fill_zero_buf(zbuf):
    zv = jnp.zeros((16,), jnp.float32)

    def row(i, carry):
        for k in range(D // 16):
            zbuf[i, pl.ds(16 * k, 16)] = zv
        return carry

    lax.fori_loop(0, CHUNK, row, 0)


def _zero_slice(zbuf, sh, s):
    base = s * RPT
    for off, n in ((0, 128), (128, 128), (256, 128), (384, 128), (512, 120)):
        pltpu.sync_copy(zbuf.at[pl.ds(0, n)], sh.at[pl.ds(base + off, n)])


H = CHUNK // 4


BPB = 2 * STG
NBODY = CPW // BPB


def _agg_loop(y_hbm, src_hbm, dst_v, src_v, rows_v, gsem, stg_sem, agg_sh, w):
    def gather_start(h, r, b):
        for q in range(4):
            pltpu.async_copy(y_hbm.at[src_v.at[h, r, pl.ds(q * H, H)]],
                             rows_v.at[b, pl.ds(q * H, H)], gsem.at[b, q])

    def gather_wait(h, r, b):
        for q in range(4):
            pltpu.make_async_copy(y_hbm.at[src_v.at[h, r, pl.ds(q * H, H)]],
                                  rows_v.at[b, pl.ds(q * H, H)],
                                  gsem.at[b, q]).wait()

    def scatter(gg, j, b):
        pltpu.sync_copy(rows_v.at[b], agg_sh.at[dst_v.at[gg * BPB + j]],
                        add=True)

    def stage_issue(h, row0):
        pltpu.async_copy(src_hbm.at[pl.ds(row0, STG)], src_v.at[h],
                         stg_sem.at[h])

    def stage_wait(h):
        pltpu.make_async_copy(src_hbm.at[pl.ds(0, STG)], src_v.at[h],
                              stg_sem.at[h]).wait()

    pltpu.sync_copy(src_hbm.at[pl.ds(w * CPW, STG)], src_v.at[0])
    stage_issue(1, w * CPW + STG)
    gather_start(0, 0, 0)

    def body(gg, carry):
        for p in range(BPB // 2):
            j0, j1, j2 = 2 * p, 2 * p + 1, 2 * p + 2
            h0, r0 = divmod(j0, STG)
            h1, r1 = divmod(j1, STG)
            gather_wait(h0, r0, 0)
            gather_start(h1, r1, 1)
            scatter(gg, j0, 0)
            gather_wait(h1, r1, 1)
            if p == STG // 2 - 1:
                @pl.when(gg < NBODY - 1)
                def _():
                    stage_issue(0, w * CPW + (gg + 1) * BPB)

                stage_wait(1)
                gather_start(1, 0, 0)
            elif p == BPB // 2 - 1:
                @pl.when(gg < NBODY - 1)
                def _():
                    stage_issue(1, w * CPW + (gg + 1) * BPB + STG)
                    stage_wait(0)
                    gather_start(0, 0, 0)
            else:
                h2, r2 = divmod(j2, STG)
                gather_start(h2, r2, 0)
            scatter(gg, j1, 1)
        return carry

    lax.fori_loop(0, NBODY, body, 0)


def _spill(sh, out, c, s):
    pltpu.sync_copy(sh.at[pl.ds(s * RPT, RPT)], out.at[c, pl.ds(s * RPT, RPT)])


def _sc_agg_cnt_body(y_hbm, src_hbm, dst_hbm, ones_hbm, agg_out,
                     cnt_out, src_v, dst_v, rows_v, gsem, stg_sem, ssem,
                     agg_sh):
    c = lax.axis_index("c")
    s = lax.axis_index("s")
    w = c * NS + s

    _fill_zero_buf(rows_v.at[1])
    _zero_slice(rows_v.at[1], agg_sh, s)
    pltpu.sync_copy(dst_hbm.at[pl.ds(w * CPW, CPW)], dst_v)
    pltpu.sync_copy(ones_hbm, rows_v.at[0])
    plsc.subcore_barrier()

    def fire(j, carry):
        pltpu.async_copy(rows_v.at[0], agg_sh.at[dst_v.at[j]], ssem, add=True)
        return carry

    lax.fori_loop(0, CPW, fire, 0)

    def drain(j, carry):
        pltpu.make_async_copy(rows_v.at[0], agg_sh.at[dst_v.at[j]],
                              ssem).wait()
        return carry

    lax.fori_loop(0, CPW, drain, 0)
    plsc.subcore_barrier()
    _spill(agg_sh, cnt_out, c, s)
    _zero_slice(rows_v.at[1], agg_sh, s)
    plsc.subcore_barrier()

    _agg_loop(y_hbm, src_hbm, dst_v, src_v, rows_v, gsem, stg_sem, agg_sh, w)
    plsc.subcore_barrier()
    _spill(agg_sh, agg_out, c, s)


def _sc_agg_body(y_hbm, src_hbm, dst_hbm, agg_out,
                 src_v, dst_v, rows_v, gsem, stg_sem, agg_sh):
    c = lax.axis_index("c")
    s = lax.axis_index("s")
    w = c * NS + s

    _fill_zero_buf(rows_v.at[1])
    _zero_slice(rows_v.at[1], agg_sh, s)
    pltpu.sync_copy(dst_hbm.at[pl.ds(w * CPW, CPW)], dst_v)
    plsc.subcore_barrier()

    _agg_loop(y_hbm, src_hbm, dst_v, src_v, rows_v, gsem, stg_sem, agg_sh, w)
    plsc.subcore_barrier()
    _spill(agg_sh, agg_out, c, s)


_sc_pass_counts = pl.kernel(
    _sc_agg_cnt_body,
    out_type=(jax.ShapeDtypeStruct((NC, N_PAD, D), jnp.float32),
              jax.ShapeDtypeStruct((NC, N_PAD, D), jnp.float32)),
    mesh=_mesh,
    scratch_types=[
        pltpu.VMEM((2, STG, CHUNK), jnp.int32),
        pltpu.VMEM((CPW, CHUNK), jnp.int32),
        pltpu.VMEM((2, CHUNK, D), jnp.float32),
        pltpu.SemaphoreType.DMA((2, 4)),
        pltpu.SemaphoreType.DMA((2,)),
        pltpu.SemaphoreType.DMA,
        pltpu.VMEM_SHARED((N_PAD, D), jnp.float32),
    ],
)

_sc_pass = pl.kernel(
    _sc_agg_body,
    out_type=jax.ShapeDtypeStruct((NC, N_PAD, D), jnp.float32),
    mesh=_mesh,
    scratch_types=[
        pltpu.VMEM((2, STG, CHUNK), jnp.int32),
        pltpu.VMEM((CPW, CHUNK), jnp.int32),
        pltpu.VMEM((2, CHUNK, D), jnp.float32),
        pltpu.SemaphoreType.DMA((2, 4)),
        pltpu.SemaphoreType.DMA((2,)),
        pltpu.VMEM_SHARED((N_PAD, D), jnp.float32),
    ],
)


R = 2000
GRID = N // R


def _lin_body(x_ref, w_ref, o_ref):
    o_ref[...] = jnp.dot(x_ref[...], w_ref[...],
                         preferred_element_type=jnp.float32)


_linear = pl.pallas_call(
    _lin_body,
    grid=(GRID,),
    in_specs=[pl.BlockSpec((R, D), lambda i: (i, 0)),
              pl.BlockSpec((D, D), lambda i: (0, 0))],
    out_specs=pl.BlockSpec((R, D), lambda i: (i, 0)),
    out_shape=jax.ShapeDtypeStruct((N, D), jnp.float32),
)


def _mean_rows(agg_ref, cnt_ref):
    cnt = cnt_ref[0, :, 0:1] + cnt_ref[1, :, 0:1]
    inv = 1.0 / jnp.maximum(cnt, 1.0)
    return (agg_ref[0] + agg_ref[1]) * inv


def _combine_body(h_ref, agg_ref, cnt_ref, wrt_ref, bl_ref, wltn_ref,
                  h_out, y_out):
    o = (_mean_rows(agg_ref, cnt_ref) + bl_ref[...]
         + jnp.dot(h_ref[...], wrt_ref[...], preferred_element_type=jnp.float32))
    hn = jnp.maximum(o, 0.0)
    h_out[...] = hn
    y_out[...] = jnp.dot(hn, wltn_ref[...], preferred_element_type=jnp.float32)


def _combine_final_body(h_ref, agg_ref, cnt_ref, wrt_ref, bl_ref, o_ref):
    o = (_mean_rows(agg_ref, cnt_ref) + bl_ref[...]
         + jnp.dot(h_ref[...], wrt_ref[...], preferred_element_type=jnp.float32))
    mx = jnp.max(o, axis=-1, keepdims=True)
    lse = jnp.log(jnp.sum(jnp.exp(o - mx), axis=-1, keepdims=True)) + mx
    o_ref[...] = o - lse


_in_specs_combine = [
    pl.BlockSpec((R, D), lambda i: (i, 0)),
    pl.BlockSpec((NC, R, D), lambda i: (0, i, 0)),
    pl.BlockSpec((NC, R, D), lambda i: (0, i, 0)),
    pl.BlockSpec((D, D), lambda i: (0, 0)),
    pl.BlockSpec((1, D), lambda i: (0, 0)),
]

_combine = pl.pallas_call(
    _combine_body,
    grid=(GRID,),
    in_specs=_in_specs_combine + [pl.BlockSpec((D, D), lambda i: (0, 0))],
    out_specs=(pl.BlockSpec((R, D), lambda i: (i, 0)),
               pl.BlockSpec((R, D), lambda i: (i, 0))),
    out_shape=(jax.ShapeDtypeStruct((N, D), jnp.float32),
               jax.ShapeDtypeStruct((N, D), jnp.float32)),
)

_combine_final = pl.pallas_call(
    _combine_final_body,
    grid=(GRID,),
    in_specs=_in_specs_combine,
    out_specs=pl.BlockSpec((R, D), lambda i: (i, 0)),
    out_shape=jax.ShapeDtypeStruct((N, D), jnp.float32),
)


def kernel(x, edge_index, Wl1, bl1, Wr1, Wl2, bl2, Wr2, Wl3, bl3, Wr3):
    src = edge_index[0].astype(jnp.int32)
    dst = edge_index[1].astype(jnp.int32)
    npad = E_PAD - E
    pad_src = (jnp.arange(npad, dtype=jnp.int32) % N)
    pad_dst = N + (jnp.arange(npad, dtype=jnp.int32) % (N_PAD - N))
    srcp = jnp.concatenate([src, pad_src]).reshape(-1, CHUNK)
    dstp = jnp.concatenate([dst, pad_dst]).reshape(-1, CHUNK)
    ones128 = jnp.ones((CHUNK, D), jnp.float32)

    y1 = _linear(x, Wl1.T)
    agg1, cntp = _sc_pass_counts(y1, srcp, dstp, ones128)
    h1, y2 = _combine(x, agg1, cntp, Wr1.T, bl1.reshape(1, D), Wl2.T)
    agg2 = _sc_pass(y2, srcp, dstp)
    h2, y3 = _combine(h1, agg2, cntp, Wr2.T, bl2.reshape(1, D), Wl3.T)
    agg3 = _sc_pass(y3, srcp, dstp)
    return _combine_final(h2, agg3, cntp, Wr3.T, bl3.reshape(1, D))
